# baseline (speedup 1.0000x reference)
import jax
import jax.numpy as jnp
from jax.experimental import pallas as pl
from jax.experimental.pallas import tpu as pltpu

N = 8192
M = 2048
STEPS = 16
ROWS = N // STEPS
WR = ROWS // 8


def _mask_kernel(rp_ref, crs_ref, crc_ref, sa_ref, xa_ref):
    rp = rp_ref[...]
    k8 = jnp.uint32(0x88888888)
    k1 = jnp.uint32(0x11111111)
    xs = rp ^ crs_ref[...]
    sa_ref[...] = pltpu.bitcast(((k8 - xs) >> 3) & k1, jnp.int4)
    xc = rp ^ crc_ref[...]
    xa_ref[...] = pltpu.bitcast(((k8 - xc) >> 3) & k1, jnp.int4)


def _nibble_pack(ids):
    lo = ids[0::2].astype(jnp.uint8) | (ids[1::2].astype(jnp.uint8) << 4)
    return jax.lax.bitcast_convert_type(lo.reshape(-1, 4), jnp.uint32)


def kernel(seq_flat, ctx_flat, seq_ids, ctx_ids):
    rp = _nibble_pack(seq_ids).reshape(N // 8, 1)
    rep = jnp.uint32(0x11111111)
    colrep_s = (seq_ids.astype(jnp.uint32) * rep).reshape(1, N)
    colrep_c = (ctx_ids.astype(jnp.uint32) * rep).reshape(1, M)

    sa_w, xa_w = pl.pallas_call(
        _mask_kernel,
        grid=(STEPS,),
        in_specs=[
            pl.BlockSpec((WR, 1), lambda i: (i, 0)),
            pl.BlockSpec((1, N), lambda i: (0, 0)),
            pl.BlockSpec((1, M), lambda i: (0, 0)),
        ],
        out_specs=[
            pl.BlockSpec((ROWS, N), lambda i: (i, 0)),
            pl.BlockSpec((ROWS, M), lambda i: (i, 0)),
        ],
        out_shape=[
            jax.ShapeDtypeStruct((N, N), jnp.int4),
            jax.ShapeDtypeStruct((N, M), jnp.int4),
        ],
    )(rp, colrep_s, colrep_c)
    return (seq_flat, ctx_flat,
            sa_w.astype(jnp.bool_), xa_w.astype(jnp.bool_))


# speedup vs baseline: 1.0158x; 1.0158x over previous
import jax
import jax.numpy as jnp
from jax.experimental import pallas as pl
from jax.experimental.pallas import tpu as pltpu

N = 8192
M = 2048
STEPS = 16
ROWS = N // STEPS
WR = ROWS // 4


def _mask_kernel(seq_i, ctx_i, rp_ref, crs_ref, crc_ref,
                 seq_o, ctx_o, sa_ref, xa_ref):
    seq_o[...] = seq_i[...]
    ctx_o[...] = ctx_i[...]
    rp = rp_ref[...]
    k80 = jnp.uint32(0x80808080)
    k01 = jnp.uint32(0x01010101)
    xs = rp ^ crs_ref[...]
    sa_ref[...] = pltpu.bitcast(((k80 - xs) >> 7) & k01, jnp.int8)
    xc = rp ^ crc_ref[...]
    xa_ref[...] = pltpu.bitcast(((k80 - xc) >> 7) & k01, jnp.int8)


def kernel(seq_flat, ctx_flat, seq_ids, ctx_ids):
    rp = jax.lax.bitcast_convert_type(
        seq_ids.astype(jnp.uint8).reshape(N // 4, 4), jnp.uint32
    ).reshape(N // 4, 1)
    rep = jnp.uint32(0x01010101)
    colrep_s = (seq_ids.astype(jnp.uint32) * rep).reshape(1, N)
    colrep_c = (ctx_ids.astype(jnp.uint32) * rep).reshape(1, M)

    seq_p, ctx_p, sa_w, xa_w = pl.pallas_call(
        _mask_kernel,
        grid=(STEPS,),
        in_specs=[
            pl.BlockSpec((1, N // STEPS, 1024), lambda i: (0, i, 0)),
            pl.BlockSpec((1, M // STEPS, 1024), lambda i: (0, i, 0)),
            pl.BlockSpec((WR, 1), lambda i: (i, 0)),
            pl.BlockSpec((1, N), lambda i: (0, 0)),
            pl.BlockSpec((1, M), lambda i: (0, 0)),
        ],
        out_specs=[
            pl.BlockSpec((1, N // STEPS, 1024), lambda i: (0, i, 0)),
            pl.BlockSpec((1, M // STEPS, 1024), lambda i: (0, i, 0)),
            pl.BlockSpec((ROWS, N), lambda i: (i, 0)),
            pl.BlockSpec((ROWS, M), lambda i: (i, 0)),
        ],
        out_shape=[
            jax.ShapeDtypeStruct((1, N, 1024), jnp.float32),
            jax.ShapeDtypeStruct((1, M, 1024), jnp.float32),
            jax.ShapeDtypeStruct((N, N), jnp.int8),
            jax.ShapeDtypeStruct((N, M), jnp.int8),
        ],
    )(seq_flat, ctx_flat, rp, colrep_s, colrep_c)
    return (seq_p, ctx_p,
            sa_w.astype(jnp.bool_), xa_w.astype(jnp.bool_))
